# baseline (speedup 1.0000x reference)
import functools

import jax
import jax.numpy as jnp
from jax import lax
from jax.experimental import pallas as pl
from jax.experimental.pallas import tpu as pltpu
from jax.experimental.pallas import tpu_sc as plsc

_B = 64
_T = 128
_D = 768
_V = 50257
_VD = _V * _D
_NB = _B * _T
_NC = 2
_NS = 16
_NW = _NC * _NS
_B_PER_W = _NB // _NW
_GCHUNK = 32
_NG = _B_PER_W // _GCHUNK
_Q = _VD // _NW
_CC = 32768
_NCC_FULL = _Q // _CC
_CC_LAST = _Q - _NCC_FULL * _CC
_NCC = _NCC_FULL + 1


@functools.partial(
    pl.kernel,
    mesh=plsc.VectorSubcoreMesh(core_axis_name="c", subcore_axis_name="s"),
    out_type=(
        jax.ShapeDtypeStruct((_VD,), jnp.float32),
        jax.ShapeDtypeStruct((_NB, _D), jnp.float32),
    ),
    scratch_types=[
        pltpu.VMEM((_NG, _GCHUNK), jnp.int32),
        pltpu.VMEM((_GCHUNK, _D), jnp.float32),
        pltpu.VMEM((_GCHUNK, _D), jnp.float32),
        pltpu.VMEM((_CC,), jnp.float32),
        pltpu.VMEM((_CC,), jnp.float32),
        pltpu.SemaphoreType.DMA,
        pltpu.SemaphoreType.DMA,
    ],
)
def _sc_copy_and_gather(
    table_hbm, flat_hbm, idx_hbm, copy_hbm, out_hbm,
    idx_v, gbuf0, gbuf1, cbuf0, cbuf1, sem0, sem1,
):
    wid = lax.axis_index("s") * _NC + lax.axis_index("c")
    gbase = wid * _B_PER_W
    cbase = wid * _Q

    pltpu.sync_copy(idx_hbm.at[wid], idx_v)
    gbufs = (gbuf0, gbuf1)
    sems = (sem0, sem1)

    def gstart(i):
        return pltpu.async_copy(table_hbm.at[idx_v.at[i]], gbufs[i % 2], sems[i % 2])

    cur = gstart(0)
    for i in range(_NG):
        nxt = gstart(i + 1) if i + 1 < _NG else None
        cur.wait()
        pltpu.sync_copy(gbufs[i % 2], out_hbm.at[pl.ds(gbase + i * _GCHUNK, _GCHUNK)])
        cur = nxt

    cbufs = (cbuf0, cbuf1)

    def cstart(i, b, size=_CC):
        return pltpu.async_copy(
            flat_hbm.at[pl.ds(cbase + i * _CC, size)],
            cbufs[b].at[pl.ds(0, size)],
            sems[b],
        )

    def cwait(b, size=_CC):
        pltpu.make_async_copy(
            flat_hbm.at[pl.ds(0, size)],
            cbufs[b].at[pl.ds(0, size)],
            sems[b],
        ).wait()

    def cwrite(i, b, size=_CC):
        pltpu.sync_copy(
            cbufs[b].at[pl.ds(0, size)],
            copy_hbm.at[pl.ds(cbase + i * _CC, size)],
        )

    cstart(0, 0)
    cstart(1, 1)

    def loop_body(g, carry):
        for b in range(2):
            i = 2 * g + b
            cwait(b)
            cwrite(i, b)
            cstart(i + 2, b)
        return carry

    lax.fori_loop(0, (_NCC_FULL - 2) // 2, loop_body, 0)

    cwait(0)
    cwrite(_NCC_FULL - 2, 0)
    cstart(_NCC_FULL, 0, size=_CC_LAST)
    cwait(1)
    cwrite(_NCC_FULL - 1, 1)
    cwait(0, size=_CC_LAST)
    cwrite(_NCC_FULL, 0, size=_CC_LAST)


def kernel(time_series_data, input_ids, word_embedding, pred_len=96, seq_len=512):
    ids = input_ids.reshape(_NW, _NG, _GCHUNK)
    flat_table = word_embedding.reshape(-1)
    copy_flat, emb = _sc_copy_and_gather(word_embedding, flat_table, ids)
    return (copy_flat.reshape(_V, _D), emb.reshape(_B, _T, _D))


# speedup vs baseline: 3.4435x; 3.4435x over previous
import functools

import jax
import jax.numpy as jnp
from jax import lax
from jax.experimental import pallas as pl
from jax.experimental.pallas import tpu as pltpu
from jax.experimental.pallas import tpu_sc as plsc

_B = 64
_T = 128
_D = 768
_NB = _B * _T
_NC = 2
_NS = 16
_NW = _NC * _NS
_B_PER_W = _NB // _NW
_CHUNK = 64
_NCHUNK = _B_PER_W // _CHUNK


@functools.partial(
    pl.kernel,
    mesh=plsc.VectorSubcoreMesh(core_axis_name="c", subcore_axis_name="s"),
    out_type=jax.ShapeDtypeStruct((_NB, _D), jnp.float32),
    scratch_types=[
        pltpu.VMEM((_NCHUNK, _CHUNK), jnp.int32),
        pltpu.VMEM((_CHUNK, _D), jnp.float32),
        pltpu.VMEM((_CHUNK, _D), jnp.float32),
        pltpu.SemaphoreType.DMA,
        pltpu.SemaphoreType.DMA,
    ],
)
def _gather_rows(table_hbm, idx_hbm, out_hbm, idx_v, buf0, buf1, sem0, sem1):
    wid = lax.axis_index("s") * _NC + lax.axis_index("c")
    base = wid * _B_PER_W
    pltpu.sync_copy(idx_hbm.at[wid], idx_v)

    bufs = (buf0, buf1)
    sems = (sem0, sem1)

    def start(c):
        return pltpu.async_copy(
            table_hbm.at[idx_v.at[c]], bufs[c % 2], sems[c % 2]
        )

    cur = start(0)
    for c in range(_NCHUNK):
        nxt = start(c + 1) if c + 1 < _NCHUNK else None
        cur.wait()
        pltpu.sync_copy(
            bufs[c % 2], out_hbm.at[pl.ds(base + c * _CHUNK, _CHUNK)]
        )
        cur = nxt


def kernel(time_series_data, input_ids, word_embedding, pred_len=96, seq_len=512):
    ids = input_ids.reshape(_NW, _NCHUNK, _CHUNK)
    flat = _gather_rows(word_embedding, ids)
    zero = time_series_data[0, 0, 0] * 0.0
    return (word_embedding + zero, flat.reshape(_B, _T, _D))


# speedup vs baseline: 3.4436x; 1.0000x over previous
import functools

import jax
import jax.numpy as jnp
from jax import lax
from jax.experimental import pallas as pl
from jax.experimental.pallas import tpu as pltpu
from jax.experimental.pallas import tpu_sc as plsc

_B = 64
_T = 128
_D = 768
_NB = _B * _T
_NC = 2
_NS = 16
_NW = _NC * _NS
_B_PER_W = _NB // _NW
_CHUNK = 16
_NCHUNK = _B_PER_W // _CHUNK
_NBUF = 8
_GWIN = 5


@functools.partial(
    pl.kernel,
    mesh=plsc.VectorSubcoreMesh(core_axis_name="c", subcore_axis_name="s"),
    out_type=jax.ShapeDtypeStruct((_NB, _D), jnp.float32),
    scratch_types=[
        pltpu.VMEM((_NCHUNK, _CHUNK), jnp.int32),
    ]
    + [pltpu.VMEM((_CHUNK, _D), jnp.float32) for _ in range(_NBUF)]
    + [pltpu.SemaphoreType.DMA for _ in range(2 * _NBUF)],
)
def _gather_rows(table_hbm, idx_hbm, out_hbm, idx_v, *bufs_and_sems):
    bufs = bufs_and_sems[:_NBUF]
    gsems = bufs_and_sems[_NBUF : 2 * _NBUF]
    wsems = bufs_and_sems[2 * _NBUF :]
    wid = lax.axis_index("s") * _NC + lax.axis_index("c")
    base = wid * _B_PER_W
    pltpu.sync_copy(idx_hbm.at[wid], idx_v)

    def gstart(c):
        b = c % _NBUF
        return pltpu.async_copy(table_hbm.at[idx_v.at[c]], bufs[b], gsems[b])

    def wstart(c):
        b = c % _NBUF
        return pltpu.async_copy(
            bufs[b], out_hbm.at[pl.ds(base + c * _CHUNK, _CHUNK)], wsems[b]
        )

    gds = {c: gstart(c) for c in range(_GWIN)}
    for c in range(_NCHUNK):
        gds[c].wait()
        wstart(c).wait()
        if c + _GWIN < _NCHUNK:
            gds[c + _GWIN] = gstart(c + _GWIN)


def kernel(time_series_data, input_ids, word_embedding, pred_len=96, seq_len=512):
    ids = input_ids.reshape(_NW, _NCHUNK, _CHUNK)
    flat = _gather_rows(word_embedding, ids)
    zero = time_series_data[0, 0, 0] * 0.0
    return (word_embedding + zero, flat.reshape(_B, _T, _D))
